# baseline (speedup 1.0000x reference)
import functools

import jax
import jax.numpy as jnp
from jax import lax
from jax.experimental import pallas as pl
from jax.experimental.pallas import tpu as pltpu
from jax.experimental.pallas import tpu_sc as plsc

_N = 10000
_E = 320000
_NSC = 2
_NSUB = 16
_NW = _NSC * _NSUB
_K = 128
_CHUNKS = 80
_EPW = _CHUNKS * _K
_EPAD = _EPW * _NW
_NPAD = 10240
_RPT = _NPAD // _NSUB
_EPS = 1e-5

@functools.lru_cache(maxsize=None)
def _sc_mesh():
    return plsc.VectorSubcoreMesh(core_axis_name="c", subcore_axis_name="s",
                                  num_cores=_NSC, num_subcores=_NSUB)


def _zero_vmem(buf, f):
    @pl.loop(0, _K)
    def _(r):
        for j in range(f // 16):
            buf[r, pl.ds(j * 16, 16)] = jnp.zeros((16,), jnp.float32)


def _hist_sc(dstp):
    @functools.partial(
        pl.kernel,
        out_type=jax.ShapeDtypeStruct((_NSC, _NPAD, 128), jnp.float32),
        mesh=_sc_mesh(),
        compiler_params=pltpu.CompilerParams(use_tc_tiling_on_sc=False),
        scratch_types=[
            pltpu.VMEM((_CHUNKS, _K), jnp.int32),
            pltpu.VMEM((_K, 16), jnp.float32),
            pltpu.VMEM_SHARED((_NPAD, 16), jnp.float32),
            pltpu.SemaphoreType.DMA,
        ],
    )
    def hist(dst_hbm, out_hbm, didx, ones, acc, semh):
        cid = lax.axis_index("c")
        sid = lax.axis_index("s")
        wid = sid * _NSC + cid
        _zero_vmem(ones, 16)
        for t in range(_RPT // _K):
            pltpu.sync_copy(ones, acc.at[pl.ds(sid * _RPT + t * _K, _K), :])

        @pl.loop(0, _K)
        def _(r):
            ones[r, pl.ds(0, 16)] = jnp.ones((16,), jnp.float32)

        pltpu.sync_copy(dst_hbm.at[wid], didx)
        plsc.subcore_barrier()

        @pl.loop(0, _CHUNKS, step=16)
        def _(c0):
            for j in range(16):
                pltpu.async_copy(ones, acc.at[didx.at[c0 + j]], semh,
                                 add=True)
            for j in range(16):
                pltpu.make_async_copy(ones, acc.at[didx.at[c0 + j]],
                                      semh).wait()

        plsc.subcore_barrier()
        pltpu.sync_copy(acc.at[pl.ds(sid * _RPT, _RPT), :],
                        out_hbm.at[cid, pl.ds(sid * _RPT, _RPT), pl.ds(0, 16)])

    return hist(dstp)


def _agg_sc(g, srcp, dstp, f):
    _NBUF = 4
    _GROUPS = _CHUNKS // _NBUF

    @functools.partial(
        pl.kernel,
        out_type=jax.ShapeDtypeStruct((_NSC, _NPAD, 128), jnp.float32),
        mesh=_sc_mesh(),
        compiler_params=pltpu.CompilerParams(use_tc_tiling_on_sc=False),
        scratch_types=[
            pltpu.VMEM((_CHUNKS, _K), jnp.int32),
            pltpu.VMEM((_CHUNKS, _K), jnp.int32),
            pltpu.VMEM((_NBUF, _K, f), jnp.float32),
            pltpu.VMEM_SHARED((_NPAD, f), jnp.float32),
            pltpu.SemaphoreType.DMA((_NBUF,)),
        ],
    )
    def agg(g_hbm, src_hbm, dst_hbm, out_hbm, sidx, didx, rows, acc, sem):
        cid = lax.axis_index("c")
        sid = lax.axis_index("s")
        wid = sid * _NSC + cid
        _zero_vmem(rows.at[0], f)
        for t in range(_RPT // _K):
            pltpu.sync_copy(rows.at[0],
                            acc.at[pl.ds(sid * _RPT + t * _K, _K), :])
        pltpu.sync_copy(src_hbm.at[wid], sidx)
        pltpu.sync_copy(dst_hbm.at[wid], didx)
        plsc.subcore_barrier()

        for b in range(_NBUF):
            pltpu.async_copy(g_hbm.at[sidx.at[b]], rows.at[b], sem.at[b])

        @pl.loop(0, _GROUPS - 1)
        def _(gb):
            c0 = gb * _NBUF
            for b in range(_NBUF):
                c = c0 + b
                pltpu.make_async_copy(g_hbm.at[sidx.at[c]], rows.at[b],
                                      sem.at[b]).wait()
                pltpu.sync_copy(rows.at[b], acc.at[didx.at[c]], add=True)
                pltpu.async_copy(g_hbm.at[sidx.at[c + _NBUF]], rows.at[b],
                                 sem.at[b])

        for b in range(_NBUF):
            c = (_GROUPS - 1) * _NBUF + b
            pltpu.make_async_copy(g_hbm.at[sidx.at[c]], rows.at[b],
                                  sem.at[b]).wait()
            pltpu.sync_copy(rows.at[b], acc.at[didx.at[c]], add=True)

        plsc.subcore_barrier()
        pltpu.sync_copy(acc.at[pl.ds(sid * _RPT, _RPT), :],
                        out_hbm.at[cid, pl.ds(sid * _RPT, _RPT), pl.ds(0, f)])

    return agg(g, srcp, dstp)


_NB = 10
_RB = _N // _NB


def _stage_ln_mm(x, ln_g, ln_b, w1):
    def body(x_ref, g_ref, b_ref, w_ref, h1_ref):
        xv = x_ref[...]
        m = jnp.mean(xv, axis=1, keepdims=True)
        v = jnp.mean((xv - m) ** 2, axis=1, keepdims=True)
        xn = (xv - m) * lax.rsqrt(v + _EPS) * g_ref[...] + b_ref[...]
        h1_ref[...] = jnp.dot(xn, w_ref[...],
                              preferred_element_type=jnp.float32)

    return pl.pallas_call(
        body,
        grid=(_NB,),
        in_specs=[
            pl.BlockSpec((_RB, 128), lambda i: (i, 0)),
            pl.BlockSpec((128,), lambda i: (0,)),
            pl.BlockSpec((128,), lambda i: (0,)),
            pl.BlockSpec((128, 96), lambda i: (0, 0)),
        ],
        out_specs=pl.BlockSpec((_RB, 96), lambda i: (i, 0)),
        out_shape=jax.ShapeDtypeStruct((_N, 96), jnp.float32),
    )(x, ln_g, ln_b, w1)


def _stage_scale(h1, histp):
    def body(h1_ref, h_ref, g1_ref, dinv_ref):
        deg = h_ref[0, :, 0:1] + h_ref[1, :, 0:1] + 1.0
        dinv = lax.rsqrt(deg)
        g1_ref[...] = h1_ref[...] * dinv
        dinv_ref[...] = dinv

    return pl.pallas_call(
        body,
        grid=(_NB,),
        in_specs=[
            pl.BlockSpec((_RB, 96), lambda i: (i, 0)),
            pl.BlockSpec((2, _RB, 128), lambda i: (0, i, 0)),
        ],
        out_specs=(pl.BlockSpec((_RB, 96), lambda i: (i, 0)),
                   pl.BlockSpec((_RB, 1), lambda i: (i, 0))),
        out_shape=(jax.ShapeDtypeStruct((_N, 96), jnp.float32),
                   jax.ShapeDtypeStruct((_N, 1), jnp.float32)),
    )(h1, histp)


def _stage_mid(aggp, gprev, dinv, bias, bn_g, bn_b, wn, fr, fout, pad_to):
    fprev = gprev.shape[1]

    def body(a_ref, g_ref, d_ref, b_ref, bg_ref, bb_ref, w_ref, o_ref,
             pre_sc, stat_sc):
        p = pl.program_id(0)
        i = pl.program_id(1)

        @pl.when(jnp.logical_and(p == 0, i == 0))
        def _():
            stat_sc[...] = jnp.zeros_like(stat_sc)

        @pl.when(p == 0)
        def _():
            s = a_ref[0, :, :fr] + a_ref[1, :, :fr] + g_ref[:, :fr]
            pre = d_ref[...] * s + b_ref[...]
            pre_sc[pl.ds(i * _RB, _RB), :] = pre
            stat_sc[0:1, :] += jnp.sum(pre, axis=0, keepdims=True)
            stat_sc[1:2, :] += jnp.sum(pre * pre, axis=0, keepdims=True)

        @pl.when(p == 1)
        def _():
            m = stat_sc[0:1, :] / float(_N)
            v = stat_sc[1:2, :] / float(_N) - m * m
            pre = pre_sc[pl.ds(i * _RB, _RB), :]
            h = (pre - m) * lax.rsqrt(v + _EPS) * bg_ref[...] + bb_ref[...]
            h = jnp.maximum(h, 0.0)
            gn = jnp.dot(h, w_ref[...],
                         preferred_element_type=jnp.float32) * d_ref[...]
            if pad_to > fout:
                gn = jnp.pad(gn, ((0, 0), (0, pad_to - fout)))
            o_ref[...] = gn

    return pl.pallas_call(
        body,
        grid=(2, _NB),
        in_specs=[
            pl.BlockSpec((2, _RB, 128),
                         lambda p, i: (0, jnp.where(p == 0, i, _NB - 1), 0)),
            pl.BlockSpec((_RB, fprev),
                         lambda p, i: (jnp.where(p == 0, i, _NB - 1), 0)),
            pl.BlockSpec((_RB, 1), lambda p, i: (i, 0)),
            pl.BlockSpec((fr,), lambda p, i: (0,)),
            pl.BlockSpec((fr,), lambda p, i: (0,)),
            pl.BlockSpec((fr,), lambda p, i: (0,)),
            pl.BlockSpec((fr, fout), lambda p, i: (0, 0)),
        ],
        out_specs=pl.BlockSpec((_RB, pad_to),
                               lambda p, i: (jnp.where(p == 1, i, 0), 0)),
        out_shape=jax.ShapeDtypeStruct((_N, pad_to), jnp.float32),
        scratch_shapes=[pltpu.VMEM((_N, fr), jnp.float32),
                        pltpu.VMEM((2, fr), jnp.float32)],
    )(aggp, gprev, dinv, bias, bn_g, bn_b, wn)


def _stage_out(aggp, gprev, dinv, b3, bn_g, bn_b, cw1, cb1, cln_g, cln_b,
               cw2, cb2):
    def body(a_ref, g_ref, d_ref, b_ref, bg_ref, bb_ref, w1_ref, c1_ref,
             lg_ref, lb_ref, w2_ref, c2_ref, o_ref, pre_sc, stat_sc):
        p = pl.program_id(0)
        i = pl.program_id(1)

        @pl.when(jnp.logical_and(p == 0, i == 0))
        def _():
            stat_sc[...] = jnp.zeros_like(stat_sc)

        @pl.when(p == 0)
        def _():
            s = a_ref[0, :, :24] + a_ref[1, :, :24] + g_ref[:, :24]
            pre = d_ref[...] * s + b_ref[...]
            pre_sc[pl.ds(i * _RB, _RB), :] = pre
            stat_sc[0:1, :] += jnp.sum(pre, axis=0, keepdims=True)
            stat_sc[1:2, :] += jnp.sum(pre * pre, axis=0, keepdims=True)

        @pl.when(p == 1)
        def _():
            m = stat_sc[0:1, :] / float(_N)
            v = stat_sc[1:2, :] / float(_N) - m * m
            pre = pre_sc[pl.ds(i * _RB, _RB), :]
            h = (pre - m) * lax.rsqrt(v + _EPS) * bg_ref[...] + bb_ref[...]
            h = jnp.maximum(h, 0.0)
            h = jnp.dot(h, w1_ref[...], preferred_element_type=jnp.float32)
            h = h + c1_ref[...]
            m2 = jnp.mean(h, axis=1, keepdims=True)
            v2 = jnp.mean((h - m2) ** 2, axis=1, keepdims=True)
            h = (h - m2) * lax.rsqrt(v2 + _EPS) * lg_ref[...] + lb_ref[...]
            h = jnp.maximum(h, 0.0)
            out = jnp.dot(h, w2_ref[...], preferred_element_type=jnp.float32)
            o_ref[...] = out + c2_ref[...]

    return pl.pallas_call(
        body,
        grid=(2, _NB),
        in_specs=[
            pl.BlockSpec((2, _RB, 128),
                         lambda p, i: (0, jnp.where(p == 0, i, _NB - 1), 0)),
            pl.BlockSpec((_RB, 32),
                         lambda p, i: (jnp.where(p == 0, i, _NB - 1), 0)),
            pl.BlockSpec((_RB, 1), lambda p, i: (i, 0)),
            pl.BlockSpec((24,), lambda p, i: (0,)),
            pl.BlockSpec((24,), lambda p, i: (0,)),
            pl.BlockSpec((24,), lambda p, i: (0,)),
            pl.BlockSpec((24, 12), lambda p, i: (0, 0)),
            pl.BlockSpec((12,), lambda p, i: (0,)),
            pl.BlockSpec((12,), lambda p, i: (0,)),
            pl.BlockSpec((12,), lambda p, i: (0,)),
            pl.BlockSpec((12, 8), lambda p, i: (0, 0)),
            pl.BlockSpec((8,), lambda p, i: (0,)),
        ],
        out_specs=pl.BlockSpec((_RB, 8),
                               lambda p, i: (jnp.where(p == 1, i, 0), 0)),
        out_shape=jax.ShapeDtypeStruct((_N, 8), jnp.float32),
        scratch_shapes=[pltpu.VMEM((_N, 24), jnp.float32),
                        pltpu.VMEM((2, 24), jnp.float32)],
    )(aggp, gprev, dinv, b3, bn_g, bn_b, cw1, cb1, cln_g, cln_b, cw2, cb2)


def kernel(x, edge_index, ln_in_g, ln_in_b, W1, b1, bn1_g, bn1_b, W2, b2,
           bn2_g, bn2_b, W3, b3, bn3_g, bn3_b, cW1, cb1, cln_g, cln_b,
           cW2, cb2):
    src, dst = edge_index[0], edge_index[1]
    npad = _EPAD - _E
    ar = jnp.arange(npad, dtype=jnp.int32)
    pad_src = (ar * 97) % _N
    pad_dst = _N + ar % (_NPAD - _N)
    srcp = jnp.concatenate([src, pad_src]).reshape(_NW, _CHUNKS, _K)
    dstp = jnp.concatenate([dst, pad_dst]).reshape(_NW, _CHUNKS, _K)

    histp = _hist_sc(dstp)
    h1 = _stage_ln_mm(x, ln_in_g, ln_in_b, W1)
    g1, dinv = _stage_scale(h1, histp)
    agg1 = _agg_sc(g1, srcp, dstp, 96)
    g2 = _stage_mid(agg1, g1, dinv, b1, bn1_g, bn1_b, W2, 96, 48, 48)
    agg2 = _agg_sc(g2, srcp, dstp, 48)
    g3 = _stage_mid(agg2, g2, dinv, b2, bn2_g, bn2_b, W3, 48, 24, 32)
    agg3 = _agg_sc(g3, srcp, dstp, 32)
    return _stage_out(agg3, g3, dinv, b3, bn3_g, bn3_b, cW1, cb1,
                      cln_g, cln_b, cW2, cb2)


# speedup vs baseline: 1.0728x; 1.0728x over previous
import functools

import jax
import jax.numpy as jnp
from jax import lax
from jax.experimental import pallas as pl
from jax.experimental.pallas import tpu as pltpu
from jax.experimental.pallas import tpu_sc as plsc

_N = 10000
_E = 320000
_NSC = 2
_NSUB = 16
_NW = _NSC * _NSUB
_K = 128
_CHUNKS = 80
_EPW = _CHUNKS * _K
_EPAD = _EPW * _NW
_NPAD = 10240
_RPT = _NPAD // _NSUB
_EPS = 1e-5

@functools.lru_cache(maxsize=None)
def _sc_mesh():
    return plsc.VectorSubcoreMesh(core_axis_name="c", subcore_axis_name="s",
                                  num_cores=_NSC, num_subcores=_NSUB)


def _zero_vmem(buf, f):
    @pl.loop(0, _K)
    def _(r):
        for j in range(f // 16):
            buf[r, pl.ds(j * 16, 16)] = jnp.zeros((16,), jnp.float32)


def _hist_sc(dstp):
    @functools.partial(
        pl.kernel,
        out_type=jax.ShapeDtypeStruct((_NSC, _NPAD, 128), jnp.float32),
        mesh=_sc_mesh(),
        compiler_params=pltpu.CompilerParams(use_tc_tiling_on_sc=False),
        scratch_types=[
            pltpu.VMEM((_CHUNKS, _K), jnp.int32),
            pltpu.VMEM((_K, 16), jnp.float32),
            pltpu.VMEM_SHARED((_NPAD, 16), jnp.float32),
            pltpu.SemaphoreType.DMA,
        ],
    )
    def hist(dst_hbm, out_hbm, didx, ones, acc, semh):
        cid = lax.axis_index("c")
        sid = lax.axis_index("s")
        wid = sid * _NSC + cid
        _zero_vmem(ones, 16)
        for t in range(_RPT // _K):
            pltpu.sync_copy(ones, acc.at[pl.ds(sid * _RPT + t * _K, _K), :])

        @pl.loop(0, _K)
        def _(r):
            ones[r, pl.ds(0, 16)] = jnp.ones((16,), jnp.float32)

        pltpu.sync_copy(dst_hbm.at[wid], didx)
        plsc.subcore_barrier()

        @pl.loop(0, _CHUNKS, step=16)
        def _(c0):
            for j in range(16):
                pltpu.async_copy(ones, acc.at[didx.at[c0 + j]], semh,
                                 add=True)
            for j in range(16):
                pltpu.make_async_copy(ones, acc.at[didx.at[c0 + j]],
                                      semh).wait()

        plsc.subcore_barrier()
        pltpu.sync_copy(acc.at[pl.ds(sid * _RPT, _RPT), :],
                        out_hbm.at[cid, pl.ds(sid * _RPT, _RPT), pl.ds(0, 16)])

    return hist(dstp)


def _agg_sc(g, srcp, dstp, f):
    _NBUF = 4
    _GROUPS = _CHUNKS // _NBUF

    @functools.partial(
        pl.kernel,
        out_type=jax.ShapeDtypeStruct((_NSC, _NPAD, 128), jnp.float32),
        mesh=_sc_mesh(),
        compiler_params=pltpu.CompilerParams(use_tc_tiling_on_sc=False),
        scratch_types=[
            pltpu.VMEM((_CHUNKS, _K), jnp.int32),
            pltpu.VMEM((_CHUNKS, _K), jnp.int32),
            pltpu.VMEM((_NBUF, _K, f), jnp.float32),
            pltpu.VMEM_SHARED((_NPAD, f), jnp.float32),
            pltpu.SemaphoreType.DMA((_NBUF,)),
        ],
    )
    def agg(g_hbm, src_hbm, dst_hbm, out_hbm, sidx, didx, rows, acc, sem):
        cid = lax.axis_index("c")
        sid = lax.axis_index("s")
        wid = sid * _NSC + cid
        _zero_vmem(rows.at[0], f)
        for t in range(_RPT // _K):
            pltpu.sync_copy(rows.at[0],
                            acc.at[pl.ds(sid * _RPT + t * _K, _K), :])
        pltpu.sync_copy(src_hbm.at[wid], sidx)
        pltpu.sync_copy(dst_hbm.at[wid], didx)
        plsc.subcore_barrier()

        for b in range(_NBUF):
            pltpu.async_copy(g_hbm.at[sidx.at[b]], rows.at[b], sem.at[b])

        @pl.loop(0, _GROUPS - 1)
        def _(gb):
            c0 = gb * _NBUF
            for b in range(_NBUF):
                c = c0 + b
                pltpu.make_async_copy(g_hbm.at[sidx.at[c]], rows.at[b],
                                      sem.at[b]).wait()
                pltpu.sync_copy(rows.at[b], acc.at[didx.at[c]], add=True)
                pltpu.async_copy(g_hbm.at[sidx.at[c + _NBUF]], rows.at[b],
                                 sem.at[b])

        for b in range(_NBUF):
            c = (_GROUPS - 1) * _NBUF + b
            pltpu.make_async_copy(g_hbm.at[sidx.at[c]], rows.at[b],
                                  sem.at[b]).wait()
            pltpu.sync_copy(rows.at[b], acc.at[didx.at[c]], add=True)

        plsc.subcore_barrier()
        pltpu.sync_copy(acc.at[pl.ds(sid * _RPT, _RPT), :],
                        out_hbm.at[cid, pl.ds(sid * _RPT, _RPT), pl.ds(0, f)])

    return agg(g, srcp, dstp)


def _stage_ln_mm(x, ln_g, ln_b, w1):
    def body(x_ref, g_ref, b_ref, w_ref, h1_ref):
        xv = x_ref[...]
        m = jnp.mean(xv, axis=1, keepdims=True)
        v = jnp.mean((xv - m) ** 2, axis=1, keepdims=True)
        xn = (xv - m) * lax.rsqrt(v + _EPS) * g_ref[...] + b_ref[...]
        h1_ref[...] = jnp.dot(xn, w_ref[...],
                              preferred_element_type=jnp.float32)

    return pl.pallas_call(
        body,
        out_shape=jax.ShapeDtypeStruct((_N, 96), jnp.float32),
    )(x, ln_g, ln_b, w1)


def _stage_scale(h1, histp):
    def body(h1_ref, h_ref, g1_ref, dinv_ref):
        deg = h_ref[0, :_N, 0:1] + h_ref[1, :_N, 0:1] + 1.0
        dinv = lax.rsqrt(deg)
        g1_ref[...] = h1_ref[...] * dinv
        dinv_ref[...] = dinv

    return pl.pallas_call(
        body,
        out_shape=(jax.ShapeDtypeStruct((_N, 96), jnp.float32),
                   jax.ShapeDtypeStruct((_N, 1), jnp.float32)),
    )(h1, histp)


def _stage_mid(aggp, gprev, dinv, bias, bn_g, bn_b, wn, fr, fout, pad_to):
    def body(a_ref, g_ref, d_ref, b_ref, bg_ref, bb_ref, w_ref, o_ref):
        s = a_ref[0, :_N, :fr] + a_ref[1, :_N, :fr] + g_ref[:, :fr]
        dinv = d_ref[...]
        pre = dinv * s + b_ref[...]
        m = jnp.mean(pre, axis=0, keepdims=True)
        v = jnp.mean((pre - m) ** 2, axis=0, keepdims=True)
        h = (pre - m) * lax.rsqrt(v + _EPS) * bg_ref[...] + bb_ref[...]
        h = jnp.maximum(h, 0.0)
        gn = jnp.dot(h, w_ref[...], preferred_element_type=jnp.float32) * dinv
        if pad_to > fout:
            gn = jnp.pad(gn, ((0, 0), (0, pad_to - fout)))
        o_ref[...] = gn

    return pl.pallas_call(
        body,
        out_shape=jax.ShapeDtypeStruct((_N, pad_to), jnp.float32),
    )(aggp, gprev, dinv, bias, bn_g, bn_b, wn)


def _stage_out(aggp, gprev, dinv, b3, bn_g, bn_b, cw1, cb1, cln_g, cln_b,
               cw2, cb2):
    def body(a_ref, g_ref, d_ref, b_ref, bg_ref, bb_ref, w1_ref, c1_ref,
             lg_ref, lb_ref, w2_ref, c2_ref, o_ref):
        s = a_ref[0, :_N, :24] + a_ref[1, :_N, :24] + g_ref[:, :24]
        pre = d_ref[...] * s + b_ref[...]
        m = jnp.mean(pre, axis=0, keepdims=True)
        v = jnp.mean((pre - m) ** 2, axis=0, keepdims=True)
        h = (pre - m) * lax.rsqrt(v + _EPS) * bg_ref[...] + bb_ref[...]
        h = jnp.maximum(h, 0.0)
        h = jnp.dot(h, w1_ref[...], preferred_element_type=jnp.float32)
        h = h + c1_ref[...]
        m2 = jnp.mean(h, axis=1, keepdims=True)
        v2 = jnp.mean((h - m2) ** 2, axis=1, keepdims=True)
        h = (h - m2) * lax.rsqrt(v2 + _EPS) * lg_ref[...] + lb_ref[...]
        h = jnp.maximum(h, 0.0)
        out = jnp.dot(h, w2_ref[...], preferred_element_type=jnp.float32)
        o_ref[...] = out + c2_ref[...]

    return pl.pallas_call(
        body,
        out_shape=jax.ShapeDtypeStruct((_N, 8), jnp.float32),
    )(aggp, gprev, dinv, b3, bn_g, bn_b, cw1, cb1, cln_g, cln_b, cw2, cb2)


def kernel(x, edge_index, ln_in_g, ln_in_b, W1, b1, bn1_g, bn1_b, W2, b2,
           bn2_g, bn2_b, W3, b3, bn3_g, bn3_b, cW1, cb1, cln_g, cln_b,
           cW2, cb2):
    src, dst = edge_index[0], edge_index[1]
    npad = _EPAD - _E
    ar = jnp.arange(npad, dtype=jnp.int32)
    pad_src = (ar * 97) % _N
    pad_dst = _N + ar % (_NPAD - _N)
    srcp = jnp.concatenate([src, pad_src]).reshape(_NW, _CHUNKS, _K)
    dstp = jnp.concatenate([dst, pad_dst]).reshape(_NW, _CHUNKS, _K)

    histp = _hist_sc(dstp)
    h1 = _stage_ln_mm(x, ln_in_g, ln_in_b, W1)
    g1, dinv = _stage_scale(h1, histp)
    agg1 = _agg_sc(g1, srcp, dstp, 96)
    g2 = _stage_mid(agg1, g1, dinv, b1, bn1_g, bn1_b, W2, 96, 48, 48)
    agg2 = _agg_sc(g2, srcp, dstp, 48)
    g3 = _stage_mid(agg2, g2, dinv, b2, bn2_g, bn2_b, W3, 48, 24, 32)
    agg3 = _agg_sc(g3, srcp, dstp, 32)
    return _stage_out(agg3, g3, dinv, b3, bn3_g, bn3_b, cW1, cb1,
                      cln_g, cln_b, cW2, cb2)
